# baseline (speedup 1.0000x reference)
import functools
import math

import jax
import jax.numpy as jnp
from jax import lax
from jax.experimental import pallas as pl
from jax.experimental.pallas import tpu as pltpu
from jax.experimental.pallas import tpu_sc as plsc

HID = 64
SCALE = math.sqrt(HID)

NC = 2
NS = 16
NW = NC * NS
LANES = 16
CHUNK = 128


def _emb_body(nchunk, idx_hbm, table_hbm, out_hbm, idx_v, rows_v, gsem):
    wid = lax.axis_index("s") * NC + lax.axis_index("c")
    pltpu.sync_copy(idx_hbm.at[wid], idx_v)

    def chunk_body(j, carry):
        pltpu.async_copy(table_hbm.at[idx_v.at[j]], rows_v, gsem).wait()

        def row_body(r, c2):
            for c4 in range(HID // LANES):
                sl = pl.ds(c4 * LANES, LANES)
                rows_v[r, sl] = rows_v[r, sl] * SCALE
            return c2

        lax.fori_loop(0, CHUNK, row_body, 0)
        pltpu.sync_copy(rows_v, out_hbm.at[wid, j])
        return carry

    lax.fori_loop(0, nchunk, chunk_body, 0)


def _make_emb(nchunk, vocab):
    mesh = plsc.VectorSubcoreMesh(core_axis_name="c", subcore_axis_name="s")
    return pl.kernel(
        functools.partial(_emb_body, nchunk),
        out_type=jax.ShapeDtypeStruct((NW, nchunk, CHUNK, HID), jnp.float32),
        mesh=mesh,
        scratch_types=[
            pltpu.VMEM((nchunk, CHUNK), jnp.int32),
            pltpu.VMEM((CHUNK, HID), jnp.float32),
            pltpu.SemaphoreType.DMA,
        ],
        compiler_params=pltpu.CompilerParams(use_tc_tiling_on_sc=False),
    )


def kernel(input_ids, table):
    orig_shape = input_ids.shape
    b_total = input_ids.size
    assert b_total % (NW * CHUNK) == 0
    nchunk = b_total // (NW * CHUNK)
    ids3 = input_ids.reshape(NW, nchunk, CHUNK).astype(jnp.int32)
    out = _make_emb(nchunk, table.shape[0])(ids3, table)
    return out.reshape(*orig_shape, HID)


# speedup vs baseline: 1.1853x; 1.1853x over previous
import functools
import math

import jax
import jax.numpy as jnp
from jax import lax
from jax.experimental import pallas as pl
from jax.experimental.pallas import tpu as pltpu
from jax.experimental.pallas import tpu_sc as plsc

HID = 64
SCALE = math.sqrt(HID)

NC = 2
NS = 16
NW = NC * NS
LANES = 16
CHUNK = 128
ROW_UNROLL = 8


def _emb_body(nchunk, idx_hbm, table_hbm, out_hbm,
              idx_v, g0, g1, s0, s1, gs0, gs1, ws0, ws1):
    wid = lax.axis_index("s") * NC + lax.axis_index("c")
    pltpu.sync_copy(idx_hbm.at[wid], idx_v)
    gbuf = (g0, g1)
    sbuf = (s0, s1)
    gsem = (gs0, gs1)
    wsem = (ws0, ws1)

    def start_gather(j, b):
        pltpu.async_copy(table_hbm.at[idx_v.at[j]], gbuf[b], gsem[b])

    def wait_gather(b):
        pltpu.make_async_copy(table_hbm.at[idx_v.at[0]], gbuf[b], gsem[b]).wait()

    def start_write(j, b):
        pltpu.async_copy(sbuf[b], out_hbm.at[wid, j], wsem[b])

    def wait_write(b):
        pltpu.make_async_copy(sbuf[b], out_hbm.at[wid, 0], wsem[b]).wait()

    def scale(b):
        def row_blk(r0, carry):
            for rr in range(ROW_UNROLL):
                r = r0 * ROW_UNROLL + rr
                for c in range(HID // LANES):
                    sl = pl.ds(c * LANES, LANES)
                    sbuf[b][r, sl] = gbuf[b][r, sl] * SCALE
            return carry

        lax.fori_loop(0, CHUNK // ROW_UNROLL, row_blk, 0)

    start_gather(0, 0)
    start_gather(1, 1)

    for j in (0, 1):
        b = j
        wait_gather(b)
        scale(b)
        start_gather(j + 2, b)
        start_write(j, b)

    @pl.loop(2, nchunk - 2, step=2)
    def _(j0):
        for b in range(2):
            j = j0 + b
            wait_gather(b)
            wait_write(b)
            scale(b)
            start_gather(j + 2, b)
            start_write(j, b)

    for b in range(2):
        j = nchunk - 2 + b
        wait_gather(b)
        wait_write(b)
        scale(b)
        start_write(j, b)
    for b in range(2):
        wait_write(b)


def _make_emb(nchunk):
    mesh = plsc.VectorSubcoreMesh(core_axis_name="c", subcore_axis_name="s")
    return pl.kernel(
        functools.partial(_emb_body, nchunk),
        out_type=jax.ShapeDtypeStruct((NW, nchunk, CHUNK, HID), jnp.float32),
        mesh=mesh,
        scratch_types=[
            pltpu.VMEM((nchunk, CHUNK), jnp.int32),
            pltpu.VMEM((CHUNK, HID), jnp.float32),
            pltpu.VMEM((CHUNK, HID), jnp.float32),
            pltpu.VMEM((CHUNK, HID), jnp.float32),
            pltpu.VMEM((CHUNK, HID), jnp.float32),
            pltpu.SemaphoreType.DMA,
            pltpu.SemaphoreType.DMA,
            pltpu.SemaphoreType.DMA,
            pltpu.SemaphoreType.DMA,
        ],
        compiler_params=pltpu.CompilerParams(use_tc_tiling_on_sc=False),
    )


def kernel(input_ids, table):
    orig_shape = input_ids.shape
    b_total = input_ids.size
    assert b_total % (NW * CHUNK) == 0
    nchunk = b_total // (NW * CHUNK)
    ids3 = input_ids.reshape(NW, nchunk, CHUNK).astype(jnp.int32)
    out = _make_emb(nchunk)(ids3, table)
    return out.reshape(*orig_shape, HID)


# speedup vs baseline: 1.2043x; 1.0160x over previous
import functools
import math

import jax
import jax.numpy as jnp
from jax import lax
from jax.experimental import pallas as pl
from jax.experimental.pallas import tpu as pltpu
from jax.experimental.pallas import tpu_sc as plsc

HID = 64
SCALE = math.sqrt(HID)

NC = 2
NS = 16
NW = NC * NS
LANES = 16
SPLIT = 128
ROW_UNROLL = 8


def _emb_body(n_rows, seq, idx_hbm, table_hbm, out_hbm,
              idx_v, g0, g1, s0, s1, gs0, gs1, ws0, ws1):
    wid = lax.axis_index("s") * NC + lax.axis_index("c")
    row0 = wid * n_rows
    pltpu.sync_copy(idx_hbm.at[pl.ds(row0, n_rows)], idx_v)
    gbuf = (g0, g1)
    sbuf = (s0, s1)
    gsem = (gs0, gs1)
    wsem = (ws0, ws1)
    rem = seq - SPLIT

    def start_gather(r, b):
        pltpu.async_copy(table_hbm.at[idx_v.at[r, pl.ds(0, SPLIT)]],
                         gbuf[b].at[pl.ds(0, SPLIT)], gsem[b])
        pltpu.async_copy(table_hbm.at[idx_v.at[r, pl.ds(SPLIT, rem)]],
                         gbuf[b].at[pl.ds(SPLIT, rem)], gsem[b])

    def wait_gather(b):
        pltpu.make_async_copy(table_hbm.at[idx_v.at[0, pl.ds(0, SPLIT)]],
                              gbuf[b].at[pl.ds(0, SPLIT)], gsem[b]).wait()
        pltpu.make_async_copy(table_hbm.at[idx_v.at[0, pl.ds(SPLIT, rem)]],
                              gbuf[b].at[pl.ds(SPLIT, rem)], gsem[b]).wait()

    def start_write(r, b):
        pltpu.async_copy(sbuf[b], out_hbm.at[row0 + r], wsem[b])

    def wait_write(b):
        pltpu.make_async_copy(sbuf[b], out_hbm.at[0], wsem[b]).wait()

    def scale(b):
        def row_blk(r0, carry):
            for rr in range(ROW_UNROLL):
                r = r0 * ROW_UNROLL + rr
                for c in range(HID // LANES):
                    sl = pl.ds(c * LANES, LANES)
                    sbuf[b][r, sl] = gbuf[b][r, sl] * SCALE
            return carry

        lax.fori_loop(0, seq // ROW_UNROLL, row_blk, 0)

    start_gather(0, 0)
    start_gather(1, 1)

    for r in (0, 1):
        b = r
        wait_gather(b)
        scale(b)
        start_gather(r + 2, b)
        start_write(r, b)

    @pl.loop(2, n_rows - 2, step=2)
    def _(r0):
        for b in range(2):
            r = r0 + b
            wait_gather(b)
            wait_write(b)
            scale(b)
            start_gather(r + 2, b)
            start_write(r, b)

    for b in range(2):
        r = n_rows - 2 + b
        wait_gather(b)
        wait_write(b)
        scale(b)
        start_write(r, b)
    for b in range(2):
        wait_write(b)


def _make_emb(n_batch, seq):
    assert n_batch % NW == 0
    n_rows = n_batch // NW
    mesh = plsc.VectorSubcoreMesh(core_axis_name="c", subcore_axis_name="s")
    return pl.kernel(
        functools.partial(_emb_body, n_rows, seq),
        out_type=jax.ShapeDtypeStruct((n_batch, seq, HID), jnp.float32),
        mesh=mesh,
        scratch_types=[
            pltpu.VMEM((n_rows, seq), jnp.int32),
            pltpu.VMEM((seq, HID), jnp.float32),
            pltpu.VMEM((seq, HID), jnp.float32),
            pltpu.VMEM((seq, HID), jnp.float32),
            pltpu.VMEM((seq, HID), jnp.float32),
            pltpu.SemaphoreType.DMA,
            pltpu.SemaphoreType.DMA,
            pltpu.SemaphoreType.DMA,
            pltpu.SemaphoreType.DMA,
        ],
        compiler_params=pltpu.CompilerParams(use_tc_tiling_on_sc=False),
    )


def kernel(input_ids, table):
    n_batch, seq = input_ids.shape
    return _make_emb(n_batch, seq)(input_ids.astype(jnp.int32), table)
